# baseline (speedup 1.0000x reference)
import jax
import jax.numpy as jnp
from jax.experimental import pallas as pl
from jax.experimental.pallas import tpu as pltpu

_B, _C, _E, _R, _A = 4096, 768, 7, 8, 3
_BT = 1024


def _body(x_ref, wg_ref, wa_ref, wb_ref, o_ref):
    xb = x_ref[...].astype(jnp.bfloat16)
    logits = jnp.dot(xb, wg_ref[...].astype(jnp.bfloat16),
                     preferred_element_type=jnp.float32)
    col = jax.lax.broadcasted_iota(jnp.int32, logits.shape, 1)
    logits = jnp.where(col < _E, logits, -jnp.inf)
    m = jnp.max(logits, axis=1, keepdims=True)
    eid = jnp.min(jnp.where(logits == m, col, _E), axis=1,
                  keepdims=True)
    jcol = jax.lax.broadcasted_iota(jnp.int32, (x_ref.shape[0], _E * _R), 1)
    mask = (jcol // _R) == eid
    for a in range(_A):
        h = jnp.dot(xb, wa_ref[a].astype(jnp.bfloat16),
                    preferred_element_type=jnp.float32)
        u = jnp.where(mask, h, 0.0).astype(jnp.bfloat16)
        out = jnp.dot(u, wb_ref[a].astype(jnp.bfloat16),
                      preferred_element_type=jnp.float32)
        c = jnp.exp(out).astype(jnp.bfloat16).astype(jnp.float32)
        o_ref[a, :, :] = jnp.log(c)


def kernel(x, w_gate, lora_a, lora_b):
    wg = jnp.pad(w_gate, ((0, 0), (0, 8 - _E)))
    wa = lora_a.reshape(_A, _E * _R, _C).transpose(0, 2, 1)
    wb = lora_b.transpose(0, 1, 3, 2).reshape(_A, _E * _R, _C)
    grid = (_B // _BT,)
    return pl.pallas_call(
        _body,
        grid=grid,
        in_specs=[
            pl.BlockSpec((_BT, _C), lambda i: (i, 0)),
            pl.BlockSpec((_C, 8), lambda i: (0, 0)),
            pl.BlockSpec((_A, _C, _E * _R), lambda i: (0, 0, 0)),
            pl.BlockSpec((_A, _E * _R, _C), lambda i: (0, 0, 0)),
        ],
        out_specs=pl.BlockSpec((_A, _BT, _C), lambda i: (0, i, 0)),
        out_shape=jax.ShapeDtypeStruct((_A, _B, _C), jnp.float32),
        compiler_params=pltpu.CompilerParams(
            dimension_semantics=("parallel",),
        ),
    )(x, wg, wa, wb)


# speedup vs baseline: 1.0029x; 1.0029x over previous
import jax
import jax.numpy as jnp
from jax.experimental import pallas as pl
from jax.experimental.pallas import tpu as pltpu

_B, _C, _E, _R, _A = 4096, 768, 7, 8, 3
_BT = 1024


def _body(x_ref, wg_ref, wa_ref, wb_ref, o_ref):
    xb = x_ref[...].astype(jnp.bfloat16)
    logits = jnp.dot(xb, wg_ref[...].astype(jnp.bfloat16),
                     preferred_element_type=jnp.float32)
    col = jax.lax.broadcasted_iota(jnp.int32, logits.shape, 1)
    logits = jnp.where(col < _E, logits, -jnp.inf)
    m = jnp.max(logits, axis=1, keepdims=True)
    eid = jnp.min(jnp.where(logits == m, col, _E), axis=1,
                  keepdims=True)
    jcol = jax.lax.broadcasted_iota(jnp.int32, (x_ref.shape[0], _E * _R), 1)
    mask = (jcol // _R) == eid
    for a in range(_A):
        h = jnp.dot(xb, wa_ref[a].astype(jnp.bfloat16),
                    preferred_element_type=jnp.float32)
        u = jnp.where(mask, h.astype(jnp.bfloat16), jnp.bfloat16(0.0))
        out = jnp.dot(u, wb_ref[a].astype(jnp.bfloat16),
                      preferred_element_type=jnp.float32)
        c = jnp.exp(out).astype(jnp.bfloat16).astype(jnp.float32)
        o_ref[a, :, :] = jnp.log(c)


def kernel(x, w_gate, lora_a, lora_b):
    wg = jnp.pad(w_gate, ((0, 0), (0, 8 - _E)))
    wa = lora_a.reshape(_A, _E * _R, _C).transpose(0, 2, 1)
    wb = lora_b.transpose(0, 1, 3, 2).reshape(_A, _E * _R, _C)
    grid = (_B // _BT,)
    return pl.pallas_call(
        _body,
        grid=grid,
        in_specs=[
            pl.BlockSpec((_BT, _C), lambda i: (i, 0)),
            pl.BlockSpec((_C, 8), lambda i: (0, 0)),
            pl.BlockSpec((_A, _C, _E * _R), lambda i: (0, 0, 0)),
            pl.BlockSpec((_A, _E * _R, _C), lambda i: (0, 0, 0)),
        ],
        out_specs=pl.BlockSpec((_A, _BT, _C), lambda i: (0, i, 0)),
        out_shape=jax.ShapeDtypeStruct((_A, _B, _C), jnp.float32),
        compiler_params=pltpu.CompilerParams(
            dimension_semantics=("parallel",),
        ),
    )(x, wg, wa, wb)
